# baseline (speedup 1.0000x reference)
import functools

import jax
import jax.numpy as jnp
from jax import lax
from jax.experimental import pallas as pl
from jax.experimental.pallas import tpu as pltpu
from jax.experimental.pallas import tpu_sc as plsc

N_NODES = 10000
N_EDGES = 320000
IN_DIM = 128
HID_DIM = 256
OUT_DIM = 128
HEADS = 4

NP = 10240
E_REAL = N_EDGES + N_NODES
K = 256
NG = K // 16
CHUNKS_PER_TILE = 81
EP = 16 * CHUNKS_PER_TILE * K
ROWS_PER_TILE = NP // 16
FSZ = 64
BN = 1024
GRID_N = NP // BN


def _tc1_body(x_ref, w1_ref, wres_ref, a1a_ref, a1b_ref, bres_ref,
              tabA_ref, tabB_ref, adcA_ref, adcB_ref, res_ref):
    x = x_ref[...]
    h = jnp.dot(x, w1_ref[...], preferred_element_type=jnp.float32)
    tabA_ref[...] = h[:, :128]
    tabB_ref[...] = h[:, 128:]
    adcA_ref[...] = jnp.dot(h, a1a_ref[...], preferred_element_type=jnp.float32)
    adcB_ref[...] = jnp.dot(h, a1b_ref[...], preferred_element_type=jnp.float32)
    res_ref[...] = (jnp.dot(x, wres_ref[...], preferred_element_type=jnp.float32)
                    + bres_ref[...])


def _tc1(x_p, W1, Wres, A1a, A1b, bres2d):
    f32 = jnp.float32
    return pl.pallas_call(
        _tc1_body,
        grid=(GRID_N,),
        in_specs=[
            pl.BlockSpec((BN, IN_DIM), lambda i: (i, 0)),
            pl.BlockSpec((IN_DIM, HID_DIM), lambda i: (0, 0)),
            pl.BlockSpec((IN_DIM, OUT_DIM), lambda i: (0, 0)),
            pl.BlockSpec((HID_DIM, 4), lambda i: (0, 0)),
            pl.BlockSpec((HID_DIM, 4), lambda i: (0, 0)),
            pl.BlockSpec((1, OUT_DIM), lambda i: (0, 0)),
        ],
        out_specs=[
            pl.BlockSpec((BN, 128), lambda i: (i, 0)),
            pl.BlockSpec((BN, 128), lambda i: (i, 0)),
            pl.BlockSpec((BN, 4), lambda i: (i, 0)),
            pl.BlockSpec((BN, 4), lambda i: (i, 0)),
            pl.BlockSpec((BN, OUT_DIM), lambda i: (i, 0)),
        ],
        out_shape=[
            jax.ShapeDtypeStruct((NP, 128), f32),
            jax.ShapeDtypeStruct((NP, 128), f32),
            jax.ShapeDtypeStruct((NP, 4), f32),
            jax.ShapeDtypeStruct((NP, 4), f32),
            jax.ShapeDtypeStruct((NP, OUT_DIM), f32),
        ],
    )(x_p, W1, Wres, A1a, A1b, bres2d)


def _tc2_body(ha_ref, hb_ref, w2_ref, a2a_ref, a2b_ref,
              h2a_ref, h2b_ref, adcA_ref, adcB_ref):
    h2 = (jnp.dot(ha_ref[...], w2_ref[:128, :], preferred_element_type=jnp.float32)
          + jnp.dot(hb_ref[...], w2_ref[128:, :], preferred_element_type=jnp.float32))
    z = jnp.zeros_like(h2[:, :64])
    h2a_ref[...] = jnp.concatenate([h2[:, :64], z], axis=1)
    h2b_ref[...] = jnp.concatenate([h2[:, 64:], z], axis=1)
    adcA_ref[...] = jnp.dot(h2, a2a_ref[...], preferred_element_type=jnp.float32)
    adcB_ref[...] = jnp.dot(h2, a2b_ref[...], preferred_element_type=jnp.float32)


def _tc2(h1A, h1B, W2, A2a, A2b):
    f32 = jnp.float32
    return pl.pallas_call(
        _tc2_body,
        grid=(GRID_N,),
        in_specs=[
            pl.BlockSpec((BN, 128), lambda i: (i, 0)),
            pl.BlockSpec((BN, 128), lambda i: (i, 0)),
            pl.BlockSpec((HID_DIM, OUT_DIM), lambda i: (0, 0)),
            pl.BlockSpec((OUT_DIM, 4), lambda i: (0, 0)),
            pl.BlockSpec((OUT_DIM, 4), lambda i: (0, 0)),
        ],
        out_specs=[
            pl.BlockSpec((BN, 128), lambda i: (i, 0)),
            pl.BlockSpec((BN, 128), lambda i: (i, 0)),
            pl.BlockSpec((BN, 4), lambda i: (i, 0)),
            pl.BlockSpec((BN, 4), lambda i: (i, 0)),
        ],
        out_shape=[
            jax.ShapeDtypeStruct((NP, 128), f32),
            jax.ShapeDtypeStruct((NP, 128), f32),
            jax.ShapeDtypeStruct((NP, 4), f32),
            jax.ShapeDtypeStruct((NP, 4), f32),
        ],
    )(h1A, h1B, W2, A2a, A2b)


_MESH = dict(core_axis_name="c", subcore_axis_name="s",
             num_cores=2, num_subcores=16)
_NO_LAYOUT = pltpu.CompilerParams(needs_layout_passes=False)
_DEN = ROWS_PER_TILE * 2


@functools.lru_cache(maxsize=None)
def _make_sc_w():
    f32 = jnp.float32
    i32 = jnp.int32
    mesh = plsc.VectorSubcoreMesh(**_MESH)

    def body(sidx_h, didx_h, adcA_h, adcB_h, w_hbm, den_hbm,
             sidx, didx, adc, wq2, didx2, denb, den_sp):
        c = lax.axis_index("c")
        s = lax.axis_index("s")
        lane = lax.iota(i32, 16)
        zero16 = jnp.zeros((16,), f32)

        @pl.when(c == 0)
        def _():
            pltpu.sync_copy(adcA_h, adc)

        @pl.when(c == 1)
        def _():
            pltpu.sync_copy(adcB_h, adc)

        def zb(i, _):
            denb[pl.ds(16 * i, 16)] = zero16
            return 0
        lax.fori_loop(0, _DEN // 16, zb, 0)
        pltpu.sync_copy(denb, den_sp.at[pl.ds(s * _DEN, _DEN)])
        plsc.subcore_barrier()

        ebase = s * (CHUNKS_PER_TILE * K)

        def chunk(i, _):
            base = ebase + i * K
            pltpu.sync_copy(sidx_h.at[pl.ds(base, K)], sidx)
            pltpu.sync_copy(didx_h.at[pl.ds(base, K)], didx)

            def grp(g, _):
                off = 16 * g
                sv = sidx[pl.ds(off, 16)]
                dv = didx[pl.ds(off, 16)]
                sv4 = sv * 4
                dv4 = dv * 4
                asA = plsc.load_gather(adc, [sv4])
                asB = plsc.load_gather(adc, [sv4 + 1])
                adA = plsc.load_gather(adc, [dv4 + 2])
                adB = plsc.load_gather(adc, [dv4 + 3])
                aA = asA + adA
                aB = asB + adB
                wA = jnp.exp(jnp.maximum(aA, 0.2 * aA))
                wB = jnp.exp(jnp.maximum(aB, 0.2 * aB))
                pos = lane * 2 + 32 * g
                dv2 = dv * 2
                plsc.store_scatter(wq2, [pos], wA)
                plsc.store_scatter(wq2, [pos + 1], wB)
                plsc.store_scatter(didx2, [pos], dv2)
                plsc.store_scatter(didx2, [pos + 1], dv2 + 1)
                return 0
            lax.fori_loop(0, NG, grp, 0)

            pltpu.sync_copy(wq2, den_sp.at[didx2], add=True)
            pltpu.sync_copy(wq2, w_hbm.at[pl.ds(c * (EP * 2) + base * 2, K * 2)])
            return 0
        lax.fori_loop(0, CHUNKS_PER_TILE, chunk, 0)
        plsc.subcore_barrier()

        pltpu.sync_copy(den_sp.at[pl.ds(s * _DEN, _DEN)], denb)
        pltpu.sync_copy(denb, den_hbm.at[pl.ds(c * (NP * 2) + s * _DEN, _DEN)])

    scratch = [
        pltpu.VMEM((K,), i32),
        pltpu.VMEM((K,), i32),
        pltpu.VMEM((NP * 4,), f32),
        pltpu.VMEM((K * 2,), f32),
        pltpu.VMEM((K * 2,), i32),
        pltpu.VMEM((_DEN,), f32),
        pltpu.VMEM_SHARED((NP * 2,), f32),
    ]
    out_type = (jax.ShapeDtypeStruct((2 * EP * 2,), f32),
                jax.ShapeDtypeStruct((2 * NP * 2,), f32))
    return pl.kernel(body, out_type=out_type, mesh=mesh, scratch_types=scratch,
                     compiler_params=_NO_LAYOUT)


KM = 128
CH_M = EP // 16 // KM


@functools.lru_cache(maxsize=None)
def _make_sc_m(C_OUT, NVA, do_relu):
    f32 = jnp.float32
    i32 = jnp.int32
    NVOUT = C_OUT // 16
    mesh = plsc.VectorSubcoreMesh(**_MESH)

    def body(sidx_h, didx_h, tabA, tabB, w_hbm, den_hbm, bias_h,
             outA, outB,
             s0, s1, s2, d0, d1, d2, w0, w1, w2, grow, msg, fout, bvec, denb,
             acc_sp, p0sem, p1sem, p2sem, gsem, ssem):
        c = lax.axis_index("c")
        s = lax.axis_index("s")
        row0 = s * ROWS_PER_TILE
        zero16 = jnp.zeros((16,), f32)
        sb = (s0, s1, s2)
        db = (d0, d1, d2)
        wb = (w0, w1, w2)
        psem = (p0sem, p1sem, p2sem)
        ebase = s * (CH_M * KM)

        pltpu.sync_copy(bias_h.at[pl.ds(8 * c, 8)], bvec)

        def zrow(e, _):
            for v in range(8):
                grow[e, pl.ds(16 * v, 16)] = zero16
                msg[e, pl.ds(16 * v, 16)] = zero16
            return 0
        lax.fori_loop(0, KM, zrow, 0)
        for r in range(ROWS_PER_TILE // KM):
            pltpu.sync_copy(grow, acc_sp.at[pl.ds(row0 + r * KM, KM)])
        plsc.subcore_barrier()

        def prefetch(ci, p):
            base = ebase + ci * KM
            pltpu.async_copy(sidx_h.at[pl.ds(base, KM)], sb[p], psem[p])
            pltpu.async_copy(didx_h.at[pl.ds(base, KM)], db[p], psem[p])
            pltpu.async_copy(
                w_hbm.at[pl.ds(c * (EP * 2) + base * 2, KM * 2)],
                wb[p], psem[p])

        def drain_prefetch(ci, p):
            base = ebase + ci * KM
            pltpu.make_async_copy(
                sidx_h.at[pl.ds(base, KM)], sb[p], psem[p]).wait()
            pltpu.make_async_copy(
                didx_h.at[pl.ds(base, KM)], db[p], psem[p]).wait()
            pltpu.make_async_copy(
                w_hbm.at[pl.ds(c * (EP * 2) + base * 2, KM * 2)],
                wb[p], psem[p]).wait()

        def gather(p):
            @pl.when(c == 0)
            def _():
                pltpu.async_copy(tabA.at[sb[p]], grow, gsem)

            @pl.when(c == 1)
            def _():
                pltpu.async_copy(tabB.at[sb[p]], grow, gsem)

        def wait_gather(p):
            @pl.when(c == 0)
            def _():
                pltpu.make_async_copy(tabA.at[sb[p]], grow, gsem).wait()

            @pl.when(c == 1)
            def _():
                pltpu.make_async_copy(tabB.at[sb[p]], grow, gsem).wait()

        prefetch(0, 0)
        prefetch(1, 1)
        drain_prefetch(0, 0)
        gather(0)

        def phase(i, cc, p_cur, p_nxt, p_pre):
            wait_gather(p_cur)

            @pl.when(cc + 1 < CH_M)
            def _():
                drain_prefetch(cc + 1, p_nxt)

            @pl.when(cc > 0)
            def _():
                pltpu.make_async_copy(
                    msg, acc_sp.at[db[p_pre]], ssem).wait()

            def srow(e, _):
                e2 = jnp.full((16,), 0, i32) + 2 * e
                wa = plsc.load_gather(wb[p_cur], [e2])
                wv = plsc.load_gather(wb[p_cur], [e2 + 1])
                for v in range(8):
                    w = wa if v < NVA else wv
                    msg[e, pl.ds(16 * v, 16)] = grow[e, pl.ds(16 * v, 16)] * w
                return 0
            lax.fori_loop(0, KM, srow, 0, unroll=2)

            @pl.when(cc + 1 < CH_M)
            def _():
                gather(p_nxt)
            pltpu.async_copy(msg, acc_sp.at[db[p_cur]], ssem, add=True)

            @pl.when(cc + 2 < CH_M)
            def _():
                prefetch(cc + 2, p_pre)

        def triple(i, _):
            cc = 3 * i
            phase(i, cc, 0, 1, 2)
            phase(i, cc + 1, 1, 2, 0)
            phase(i, cc + 2, 2, 0, 1)
            return 0
        lax.fori_loop(0, CH_M // 3, triple, 0)
        pltpu.make_async_copy(
            msg, acc_sp.at[db[(CH_M - 1) % 3]], ssem).wait()
        plsc.subcore_barrier()

        def fin(half, _):
            r0 = row0 + half * FSZ
            pltpu.sync_copy(acc_sp.at[pl.ds(r0, FSZ)], grow.at[pl.ds(0, FSZ)])
            pltpu.sync_copy(
                den_hbm.at[pl.ds(c * (NP * 2) + r0 * 2, FSZ * 2)], denb)

            def frow(n, _):
                n2 = jnp.full((16,), 0, i32) + 2 * n
                da = plsc.load_gather(denb, [n2]) + 1e-16
                db = plsc.load_gather(denb, [n2 + 1]) + 1e-16
                for v in range(NVOUT):
                    d = da if v < NVA else db
                    xv = grow[n, pl.ds(16 * v, 16)] / d + bvec[0, pl.ds(16 * v, 16)]
                    if do_relu:
                        xv = jnp.maximum(xv, 0.0)
                    if C_OUT == 128:
                        grow[n, pl.ds(16 * v, 16)] = xv
                    else:
                        fout[n, pl.ds(16 * v, 16)] = xv
                return 0
            lax.fori_loop(0, FSZ, frow, 0)

            src = grow if C_OUT == 128 else fout

            @pl.when(c == 0)
            def _():
                pltpu.sync_copy(src.at[pl.ds(0, FSZ)], outA.at[pl.ds(r0, FSZ)])

            @pl.when(c == 1)
            def _():
                pltpu.sync_copy(src.at[pl.ds(0, FSZ)], outB.at[pl.ds(r0, FSZ)])
            return 0
        lax.fori_loop(0, ROWS_PER_TILE // FSZ, fin, 0)

    scratch = (
        [pltpu.VMEM((KM,), i32) for _ in range(6)]
        + [pltpu.VMEM((KM * 2,), f32) for _ in range(3)]
        + [
            pltpu.VMEM((KM, 128), f32),
            pltpu.VMEM((KM, 128), f32),
            pltpu.VMEM((FSZ, C_OUT), f32) if C_OUT != 128
            else pltpu.VMEM((8, 128), f32),
            pltpu.VMEM((8, 128), f32),
            pltpu.VMEM((FSZ * 2,), f32),
            pltpu.VMEM_SHARED((NP, 128), f32),
        ]
        + [pltpu.SemaphoreType.DMA for _ in range(5)]
    )
    out_type = (jax.ShapeDtypeStruct((NP, C_OUT), f32),
                jax.ShapeDtypeStruct((NP, C_OUT), f32))
    return pl.kernel(body, out_type=out_type, mesh=mesh, scratch_types=scratch,
                     compiler_params=_NO_LAYOUT)


def _att_mat(att_s, att_d, width, g0):
    cols = []
    for att, g in ((att_s, g0), (att_s, g0 + 1), (att_d, g0), (att_d, g0 + 1)):
        sel = jnp.zeros((HEADS, 1), att.dtype).at[g, 0].set(1.0)
        cols.append((att[:, :, None] * sel[:, None, :]).reshape(width, 1))
    return jnp.concatenate(cols, axis=1)


def kernel(x, edge_index, W1, att_src1, att_dst1, b1, W2, att_src2, att_dst2,
           b2, Wres, bres):
    i32 = jnp.int32
    loop = jnp.arange(N_NODES, dtype=i32)
    pad = jnp.full((EP - E_REAL,), N_NODES, dtype=i32)
    src = jnp.concatenate([edge_index[0].astype(i32), loop, pad])
    dst = jnp.concatenate([edge_index[1].astype(i32), loop, pad])

    x_p = jnp.pad(x, ((0, NP - N_NODES), (0, 0)))
    A1a = _att_mat(att_src1, att_dst1, HID_DIM, 0)
    A1b = _att_mat(att_src1, att_dst1, HID_DIM, 2)
    A2a = _att_mat(att_src2, att_dst2, OUT_DIM, 0)
    A2b = _att_mat(att_src2, att_dst2, OUT_DIM, 2)

    b1p = jnp.zeros((16, 128), jnp.float32)
    b1p = b1p.at[0].set(b1[:128]).at[8].set(b1[128:])
    b2p = jnp.zeros((16, 128), jnp.float32)
    b2p = b2p.at[0, :64].set(b2[:64]).at[8, :64].set(b2[64:])

    tabA, tabB, adc1A, adc1B, res = _tc1(
        x_p, W1, Wres, A1a, A1b, bres.reshape(1, OUT_DIM))
    w1, den1 = _make_sc_w()(src, dst, adc1A.reshape(-1), adc1B.reshape(-1))
    h1A, h1B = _make_sc_m(128, 4, True)(
        src, dst, tabA, tabB, w1, den1, b1p)
    h2A, h2B, adc2A, adc2B = _tc2(h1A, h1B, W2, A2a, A2b)
    w2, den2 = _make_sc_w()(src, dst, adc2A.reshape(-1), adc2B.reshape(-1))
    outA, outB = _make_sc_m(64, 2, False)(
        src, dst, h2A, h2B, w2, den2, b2p)
    gat2 = jnp.concatenate([outA[:N_NODES], outB[:N_NODES]], axis=1)
    return gat2 + res[:N_NODES]


# speedup vs baseline: 1.8310x; 1.8310x over previous
import functools

import jax
import jax.numpy as jnp
from jax import lax
from jax.experimental import pallas as pl
from jax.experimental.pallas import tpu as pltpu
from jax.experimental.pallas import tpu_sc as plsc

N_NODES = 10000
N_EDGES = 320000
IN_DIM = 128
HID_DIM = 256
OUT_DIM = 128
HEADS = 4

NP = 10240
E_REAL = N_EDGES + N_NODES
K = 288
NG = K // 16
CHUNKS_PER_TILE = 72
EP = 16 * CHUNKS_PER_TILE * K
ROWS_PER_TILE = NP // 16
FSZ = 64
BN = 1024
GRID_N = NP // BN


def _tc1_body(x_ref, w1_ref, wres_ref, a1a_ref, a1b_ref, bres_ref,
              tabA_ref, tabB_ref, adcA_ref, adcB_ref, res_ref):
    x = x_ref[...]
    h = jnp.dot(x, w1_ref[...], preferred_element_type=jnp.float32)
    tabA_ref[...] = h[:, :128]
    tabB_ref[...] = h[:, 128:]
    adcA_ref[...] = jnp.dot(h, a1a_ref[...], preferred_element_type=jnp.float32)
    adcB_ref[...] = jnp.dot(h, a1b_ref[...], preferred_element_type=jnp.float32)
    res_ref[...] = (jnp.dot(x, wres_ref[...], preferred_element_type=jnp.float32)
                    + bres_ref[...])


def _tc1(x_p, W1, Wres, A1a, A1b, bres2d):
    f32 = jnp.float32
    return pl.pallas_call(
        _tc1_body,
        grid=(GRID_N,),
        in_specs=[
            pl.BlockSpec((BN, IN_DIM), lambda i: (i, 0)),
            pl.BlockSpec((IN_DIM, HID_DIM), lambda i: (0, 0)),
            pl.BlockSpec((IN_DIM, OUT_DIM), lambda i: (0, 0)),
            pl.BlockSpec((HID_DIM, 4), lambda i: (0, 0)),
            pl.BlockSpec((HID_DIM, 4), lambda i: (0, 0)),
            pl.BlockSpec((1, OUT_DIM), lambda i: (0, 0)),
        ],
        out_specs=[
            pl.BlockSpec((BN, 128), lambda i: (i, 0)),
            pl.BlockSpec((BN, 128), lambda i: (i, 0)),
            pl.BlockSpec((BN, 4), lambda i: (i, 0)),
            pl.BlockSpec((BN, 4), lambda i: (i, 0)),
            pl.BlockSpec((BN, OUT_DIM), lambda i: (i, 0)),
        ],
        out_shape=[
            jax.ShapeDtypeStruct((NP, 128), f32),
            jax.ShapeDtypeStruct((NP, 128), f32),
            jax.ShapeDtypeStruct((NP, 4), f32),
            jax.ShapeDtypeStruct((NP, 4), f32),
            jax.ShapeDtypeStruct((NP, OUT_DIM), f32),
        ],
    )(x_p, W1, Wres, A1a, A1b, bres2d)


def _tc2_body(ha_ref, hb_ref, w2_ref, a2a_ref, a2b_ref,
              h2_ref, adcA_ref, adcB_ref):
    h2 = (jnp.dot(ha_ref[...], w2_ref[:128, :], preferred_element_type=jnp.float32)
          + jnp.dot(hb_ref[...], w2_ref[128:, :], preferred_element_type=jnp.float32))
    h2_ref[...] = h2
    adcA_ref[...] = jnp.dot(h2, a2a_ref[...], preferred_element_type=jnp.float32)
    adcB_ref[...] = jnp.dot(h2, a2b_ref[...], preferred_element_type=jnp.float32)


def _tc2(h1A, h1B, W2, A2a, A2b):
    f32 = jnp.float32
    return pl.pallas_call(
        _tc2_body,
        grid=(GRID_N,),
        in_specs=[
            pl.BlockSpec((BN, 128), lambda i: (i, 0)),
            pl.BlockSpec((BN, 128), lambda i: (i, 0)),
            pl.BlockSpec((HID_DIM, OUT_DIM), lambda i: (0, 0)),
            pl.BlockSpec((OUT_DIM, 4), lambda i: (0, 0)),
            pl.BlockSpec((OUT_DIM, 4), lambda i: (0, 0)),
        ],
        out_specs=[
            pl.BlockSpec((BN, 128), lambda i: (i, 0)),
            pl.BlockSpec((BN, 4), lambda i: (i, 0)),
            pl.BlockSpec((BN, 4), lambda i: (i, 0)),
        ],
        out_shape=[
            jax.ShapeDtypeStruct((NP, 128), f32),
            jax.ShapeDtypeStruct((NP, 4), f32),
            jax.ShapeDtypeStruct((NP, 4), f32),
        ],
    )(h1A, h1B, W2, A2a, A2b)


_MESH = dict(core_axis_name="c", subcore_axis_name="s",
             num_cores=2, num_subcores=16)
_NO_LAYOUT = pltpu.CompilerParams(needs_layout_passes=False)
_DEN = ROWS_PER_TILE * 2


@functools.lru_cache(maxsize=None)
def _make_sc_w():
    f32 = jnp.float32
    i32 = jnp.int32
    mesh = plsc.VectorSubcoreMesh(**_MESH)

    def body(sidx_h, didx_h, adcA_h, adcB_h, w_hbm, den_hbm,
             sidx, didx, adc, wq2, didx2, denb, den_sp):
        c = lax.axis_index("c")
        s = lax.axis_index("s")
        lane = lax.iota(i32, 16)
        zero16 = jnp.zeros((16,), f32)

        @pl.when(c == 0)
        def _():
            pltpu.sync_copy(adcA_h, adc)

        @pl.when(c == 1)
        def _():
            pltpu.sync_copy(adcB_h, adc)

        def zb(i, _):
            denb[pl.ds(16 * i, 16)] = zero16
            return 0
        lax.fori_loop(0, _DEN // 16, zb, 0)
        pltpu.sync_copy(denb, den_sp.at[pl.ds(s * _DEN, _DEN)])
        plsc.subcore_barrier()

        ebase = s * (CHUNKS_PER_TILE * K)

        def chunk(i, _):
            base = ebase + i * K
            pltpu.sync_copy(sidx_h.at[pl.ds(base, K)], sidx)
            pltpu.sync_copy(didx_h.at[pl.ds(base, K)], didx)

            def grp(g, _):
                off = 16 * g
                sv = sidx[pl.ds(off, 16)]
                dv = didx[pl.ds(off, 16)]
                sv4 = sv * 4
                dv4 = dv * 4
                asA = plsc.load_gather(adc, [sv4])
                asB = plsc.load_gather(adc, [sv4 + 1])
                adA = plsc.load_gather(adc, [dv4 + 2])
                adB = plsc.load_gather(adc, [dv4 + 3])
                aA = asA + adA
                aB = asB + adB
                wA = jnp.exp(jnp.maximum(aA, 0.2 * aA))
                wB = jnp.exp(jnp.maximum(aB, 0.2 * aB))
                pos = lane * 2 + 32 * g
                dv2 = dv * 2
                plsc.store_scatter(wq2, [pos], wA)
                plsc.store_scatter(wq2, [pos + 1], wB)
                plsc.store_scatter(didx2, [pos], dv2)
                plsc.store_scatter(didx2, [pos + 1], dv2 + 1)
                return 0
            lax.fori_loop(0, NG, grp, 0)

            pltpu.sync_copy(wq2, den_sp.at[didx2], add=True)
            pltpu.sync_copy(wq2, w_hbm.at[pl.ds(c * (EP * 2) + base * 2, K * 2)])
            return 0
        lax.fori_loop(0, CHUNKS_PER_TILE, chunk, 0)
        plsc.subcore_barrier()

        pltpu.sync_copy(den_sp.at[pl.ds(s * _DEN, _DEN)], denb)
        pltpu.sync_copy(denb, den_hbm.at[pl.ds(c * (NP * 2) + s * _DEN, _DEN)])

    scratch = [
        pltpu.VMEM((K,), i32),
        pltpu.VMEM((K,), i32),
        pltpu.VMEM((NP * 4,), f32),
        pltpu.VMEM((K * 2,), f32),
        pltpu.VMEM((K * 2,), i32),
        pltpu.VMEM((_DEN,), f32),
        pltpu.VMEM_SHARED((NP * 2,), f32),
    ]
    out_type = (jax.ShapeDtypeStruct((2 * EP * 2,), f32),
                jax.ShapeDtypeStruct((2 * NP * 2,), f32))
    return pl.kernel(body, out_type=out_type, mesh=mesh, scratch_types=scratch,
                     compiler_params=_NO_LAYOUT)


@functools.lru_cache(maxsize=None)
def _make_sc_m(C_OUT, NVA, do_relu):
    f32 = jnp.float32
    i32 = jnp.int32
    NVOUT = C_OUT // 16
    mesh = plsc.VectorSubcoreMesh(**_MESH)

    def body(sidx_h, didx_h, tabA, tabB, w_hbm, den_hbm, bias_h,
             outA, outB,
             sidx, didx, wq2, grow, fout, bvec, denb, acc_sp, sem):
        c = lax.axis_index("c")
        s = lax.axis_index("s")
        row0 = s * ROWS_PER_TILE
        zero16 = jnp.zeros((16,), f32)

        pltpu.sync_copy(bias_h.at[pl.ds(8 * c, 8)], bvec)

        def zrow(e, _):
            for v in range(8):
                grow[e, pl.ds(16 * v, 16)] = zero16
            return 0
        lax.fori_loop(0, K, zrow, 0)
        pltpu.sync_copy(grow, acc_sp.at[pl.ds(row0, K)])
        pltpu.sync_copy(grow, acc_sp.at[pl.ds(row0 + K, K)])
        pltpu.sync_copy(grow.at[pl.ds(0, ROWS_PER_TILE - 2 * K)],
                        acc_sp.at[pl.ds(row0 + 2 * K, ROWS_PER_TILE - 2 * K)])
        plsc.subcore_barrier()

        ebase = s * (CHUNKS_PER_TILE * K)

        def chunk(i, _):
            base = ebase + i * K
            pltpu.sync_copy(sidx_h.at[pl.ds(base, K)], sidx)
            pltpu.sync_copy(didx_h.at[pl.ds(base, K)], didx)
            pltpu.sync_copy(w_hbm.at[pl.ds(c * (EP * 2) + base * 2, K * 2)],
                            wq2)

            @pl.when(c == 0)
            def _():
                pltpu.async_copy(tabA.at[sidx], grow, sem).wait()

            @pl.when(c == 1)
            def _():
                pltpu.async_copy(tabB.at[sidx], grow, sem).wait()

            def srow(e, _):
                e2 = jnp.full((16,), 0, i32) + 2 * e
                wa = plsc.load_gather(wq2, [e2])
                wb = plsc.load_gather(wq2, [e2 + 1])
                for v in range(8):
                    w = wa if v < NVA else wb
                    grow[e, pl.ds(16 * v, 16)] = grow[e, pl.ds(16 * v, 16)] * w
                return 0
            lax.fori_loop(0, K, srow, 0)

            pltpu.sync_copy(grow, acc_sp.at[didx], add=True)
            return 0
        lax.fori_loop(0, CHUNKS_PER_TILE, chunk, 0)
        plsc.subcore_barrier()

        def fin(half, _):
            r0 = row0 + half * FSZ
            pltpu.sync_copy(acc_sp.at[pl.ds(r0, FSZ)], grow.at[pl.ds(0, FSZ)])
            pltpu.sync_copy(
                den_hbm.at[pl.ds(c * (NP * 2) + r0 * 2, FSZ * 2)], denb)

            def frow(n, _):
                n2 = jnp.full((16,), 0, i32) + 2 * n
                da = plsc.load_gather(denb, [n2]) + 1e-16
                db = plsc.load_gather(denb, [n2 + 1]) + 1e-16
                for v in range(NVOUT):
                    d = da if v < NVA else db
                    xv = grow[n, pl.ds(16 * v, 16)] / d + bvec[0, pl.ds(16 * v, 16)]
                    if do_relu:
                        xv = jnp.maximum(xv, 0.0)
                    if C_OUT == 128:
                        grow[n, pl.ds(16 * v, 16)] = xv
                    else:
                        fout[n, pl.ds(16 * v, 16)] = xv
                return 0
            lax.fori_loop(0, FSZ, frow, 0)

            src = grow if C_OUT == 128 else fout

            @pl.when(c == 0)
            def _():
                pltpu.sync_copy(src.at[pl.ds(0, FSZ)], outA.at[pl.ds(r0, FSZ)])

            @pl.when(c == 1)
            def _():
                pltpu.sync_copy(src.at[pl.ds(0, FSZ)], outB.at[pl.ds(r0, FSZ)])
            return 0
        lax.fori_loop(0, ROWS_PER_TILE // FSZ, fin, 0)

    scratch = [
        pltpu.VMEM((K,), i32),
        pltpu.VMEM((K,), i32),
        pltpu.VMEM((K * 2,), f32),
        pltpu.VMEM((K, 128), f32),
        pltpu.VMEM((FSZ, C_OUT), f32) if C_OUT != 128
        else pltpu.VMEM((8, 128), f32),
        pltpu.VMEM((8, 128), f32),
        pltpu.VMEM((FSZ * 2,), f32),
        pltpu.VMEM_SHARED((NP, 128), f32),
        pltpu.SemaphoreType.DMA,
    ]
    out_type = (jax.ShapeDtypeStruct((NP, C_OUT), f32),
                jax.ShapeDtypeStruct((NP, C_OUT), f32))
    return pl.kernel(body, out_type=out_type, mesh=mesh, scratch_types=scratch,
                     compiler_params=_NO_LAYOUT)


CH_M2 = CHUNKS_PER_TILE // 2


@functools.lru_cache(maxsize=None)
def _make_sc_m2():
    f32 = jnp.float32
    i32 = jnp.int32
    mesh = plsc.VectorSubcoreMesh(**_MESH)

    def body(sidx_h, didx_h, tab, w_hbm, outX,
             sidx, didx, wqA, wqB, grow, acc_sp, sem):
        c = lax.axis_index("c")
        s = lax.axis_index("s")
        row0 = s * ROWS_PER_TILE
        zero16 = jnp.zeros((16,), f32)

        def zrow(e, _):
            for v in range(8):
                grow[e, pl.ds(16 * v, 16)] = zero16
            return 0
        lax.fori_loop(0, K, zrow, 0)
        pltpu.sync_copy(grow, acc_sp.at[pl.ds(row0, K)])
        pltpu.sync_copy(grow, acc_sp.at[pl.ds(row0 + K, K)])
        pltpu.sync_copy(grow.at[pl.ds(0, ROWS_PER_TILE - 2 * K)],
                        acc_sp.at[pl.ds(row0 + 2 * K, ROWS_PER_TILE - 2 * K)])
        plsc.subcore_barrier()

        ebase = (c * 16 + s) * (CH_M2 * K)

        def chunk(i, _):
            base = ebase + i * K
            pltpu.sync_copy(sidx_h.at[pl.ds(base, K)], sidx)
            pltpu.sync_copy(didx_h.at[pl.ds(base, K)], didx)
            pltpu.sync_copy(w_hbm.at[pl.ds(base * 2, K * 2)], wqA)
            pltpu.sync_copy(w_hbm.at[pl.ds(EP * 2 + base * 2, K * 2)], wqB)
            pltpu.async_copy(tab.at[sidx], grow, sem).wait()

            def srow(e, _):
                e2 = jnp.full((16,), 0, i32) + 2 * e
                w0 = plsc.load_gather(wqA, [e2])
                w1 = plsc.load_gather(wqA, [e2 + 1])
                w2 = plsc.load_gather(wqB, [e2])
                w3 = plsc.load_gather(wqB, [e2 + 1])
                ws = (w0, w0, w1, w1, w2, w2, w3, w3)
                for v in range(8):
                    grow[e, pl.ds(16 * v, 16)] = (
                        grow[e, pl.ds(16 * v, 16)] * ws[v])
                return 0
            lax.fori_loop(0, K, srow, 0)

            pltpu.sync_copy(grow, acc_sp.at[didx], add=True)
            return 0
        lax.fori_loop(0, CH_M2, chunk, 0)
        plsc.subcore_barrier()

        def fin(half, _):
            r0 = row0 + half * FSZ
            pltpu.sync_copy(acc_sp.at[pl.ds(r0, FSZ)], grow.at[pl.ds(0, FSZ)])
            pltpu.sync_copy(grow.at[pl.ds(0, FSZ)],
                            outX.at[pl.ds(c * NP + r0, FSZ)])
            return 0
        lax.fori_loop(0, ROWS_PER_TILE // FSZ, fin, 0)

    scratch = [
        pltpu.VMEM((K,), i32),
        pltpu.VMEM((K,), i32),
        pltpu.VMEM((K * 2,), f32),
        pltpu.VMEM((K * 2,), f32),
        pltpu.VMEM((K, 128), f32),
        pltpu.VMEM_SHARED((NP, 128), f32),
        pltpu.SemaphoreType.DMA,
    ]
    out_type = jax.ShapeDtypeStruct((2 * NP, 128), f32)
    return pl.kernel(body, out_type=out_type, mesh=mesh, scratch_types=scratch,
                     compiler_params=_NO_LAYOUT)


def _att_mat(att_s, att_d, width, g0):
    cols = []
    for att, g in ((att_s, g0), (att_s, g0 + 1), (att_d, g0), (att_d, g0 + 1)):
        sel = jnp.zeros((HEADS, 1), att.dtype).at[g, 0].set(1.0)
        cols.append((att[:, :, None] * sel[:, None, :]).reshape(width, 1))
    return jnp.concatenate(cols, axis=1)


def kernel(x, edge_index, W1, att_src1, att_dst1, b1, W2, att_src2, att_dst2,
           b2, Wres, bres):
    i32 = jnp.int32
    loop = jnp.arange(N_NODES, dtype=i32)
    pad = jnp.full((EP - E_REAL,), N_NODES, dtype=i32)
    src = jnp.concatenate([edge_index[0].astype(i32), loop, pad])
    dst = jnp.concatenate([edge_index[1].astype(i32), loop, pad])

    x_p = jnp.pad(x, ((0, NP - N_NODES), (0, 0)))
    A1a = _att_mat(att_src1, att_dst1, HID_DIM, 0)
    A1b = _att_mat(att_src1, att_dst1, HID_DIM, 2)
    A2a = _att_mat(att_src2, att_dst2, OUT_DIM, 0)
    A2b = _att_mat(att_src2, att_dst2, OUT_DIM, 2)

    b1p = jnp.zeros((16, 128), jnp.float32)
    b1p = b1p.at[0].set(b1[:128]).at[8].set(b1[128:])

    tabA, tabB, adc1A, adc1B, res = _tc1(
        x_p, W1, Wres, A1a, A1b, bres.reshape(1, OUT_DIM))
    w1, den1 = _make_sc_w()(src, dst, adc1A.reshape(-1), adc1B.reshape(-1))
    h1A, h1B = _make_sc_m(128, 4, True)(
        src, dst, tabA, tabB, w1, den1, b1p)
    h2, adc2A, adc2B = _tc2(h1A, h1B, W2, A2a, A2b)
    w2, den2 = _make_sc_w()(src, dst, adc2A.reshape(-1), adc2B.reshape(-1))
    acc2 = _make_sc_m2()(src, dst, h2, w2)
    acc = acc2[:N_NODES] + acc2[NP:NP + N_NODES]
    den = den2.reshape(2, NP, 2)
    den_nh = jnp.concatenate([den[0], den[1]], axis=1)[:N_NODES]
    dd = jnp.repeat(den_nh + 1e-16, OUT_DIM // 4, axis=1)
    return acc / dd + b2 + res[:N_NODES]


# speedup vs baseline: 1.9303x; 1.0542x over previous
import functools

import jax
import jax.numpy as jnp
from jax import lax
from jax.experimental import pallas as pl
from jax.experimental.pallas import tpu as pltpu
from jax.experimental.pallas import tpu_sc as plsc

N_NODES = 10000
N_EDGES = 320000
IN_DIM = 128
HID_DIM = 256
OUT_DIM = 128
HEADS = 4

NP = 10240
E_REAL = N_EDGES + N_NODES
K = 288
NG = K // 16
CHUNKS_PER_TILE = 72
EP = 16 * CHUNKS_PER_TILE * K
ROWS_PER_TILE = NP // 16
FSZ = 64
BN = 1024
GRID_N = NP // BN


def _tc1_body(x_ref, w1_ref, wres_ref, a1a_ref, a1b_ref, bres_ref,
              tabA_ref, tabB_ref, adcA_ref, adcB_ref, res_ref):
    x = x_ref[...]
    h = jnp.dot(x, w1_ref[...], preferred_element_type=jnp.float32)
    tabA_ref[...] = h[:, :128]
    tabB_ref[...] = h[:, 128:]
    adcA_ref[...] = jnp.dot(h, a1a_ref[...], preferred_element_type=jnp.float32)
    adcB_ref[...] = jnp.dot(h, a1b_ref[...], preferred_element_type=jnp.float32)
    res_ref[...] = (jnp.dot(x, wres_ref[...], preferred_element_type=jnp.float32)
                    + bres_ref[...])


def _tc1(x_p, W1, Wres, A1a, A1b, bres2d):
    f32 = jnp.float32
    return pl.pallas_call(
        _tc1_body,
        grid=(GRID_N,),
        in_specs=[
            pl.BlockSpec((BN, IN_DIM), lambda i: (i, 0)),
            pl.BlockSpec((IN_DIM, HID_DIM), lambda i: (0, 0)),
            pl.BlockSpec((IN_DIM, OUT_DIM), lambda i: (0, 0)),
            pl.BlockSpec((HID_DIM, 4), lambda i: (0, 0)),
            pl.BlockSpec((HID_DIM, 4), lambda i: (0, 0)),
            pl.BlockSpec((1, OUT_DIM), lambda i: (0, 0)),
        ],
        out_specs=[
            pl.BlockSpec((BN, 128), lambda i: (i, 0)),
            pl.BlockSpec((BN, 128), lambda i: (i, 0)),
            pl.BlockSpec((BN, 4), lambda i: (i, 0)),
            pl.BlockSpec((BN, 4), lambda i: (i, 0)),
            pl.BlockSpec((BN, OUT_DIM), lambda i: (i, 0)),
        ],
        out_shape=[
            jax.ShapeDtypeStruct((NP, 128), f32),
            jax.ShapeDtypeStruct((NP, 128), f32),
            jax.ShapeDtypeStruct((NP, 4), f32),
            jax.ShapeDtypeStruct((NP, 4), f32),
            jax.ShapeDtypeStruct((NP, OUT_DIM), f32),
        ],
    )(x_p, W1, Wres, A1a, A1b, bres2d)


def _tc2_body(ha_ref, hb_ref, w2_ref, a2a_ref, a2b_ref,
              h2_ref, adcA_ref, adcB_ref):
    h2 = (jnp.dot(ha_ref[...], w2_ref[:128, :], preferred_element_type=jnp.float32)
          + jnp.dot(hb_ref[...], w2_ref[128:, :], preferred_element_type=jnp.float32))
    h2_ref[...] = h2
    adcA_ref[...] = jnp.dot(h2, a2a_ref[...], preferred_element_type=jnp.float32)
    adcB_ref[...] = jnp.dot(h2, a2b_ref[...], preferred_element_type=jnp.float32)


def _tc2(h1A, h1B, W2, A2a, A2b):
    f32 = jnp.float32
    return pl.pallas_call(
        _tc2_body,
        grid=(GRID_N,),
        in_specs=[
            pl.BlockSpec((BN, 128), lambda i: (i, 0)),
            pl.BlockSpec((BN, 128), lambda i: (i, 0)),
            pl.BlockSpec((HID_DIM, OUT_DIM), lambda i: (0, 0)),
            pl.BlockSpec((OUT_DIM, 4), lambda i: (0, 0)),
            pl.BlockSpec((OUT_DIM, 4), lambda i: (0, 0)),
        ],
        out_specs=[
            pl.BlockSpec((BN, 128), lambda i: (i, 0)),
            pl.BlockSpec((BN, 4), lambda i: (i, 0)),
            pl.BlockSpec((BN, 4), lambda i: (i, 0)),
        ],
        out_shape=[
            jax.ShapeDtypeStruct((NP, 128), f32),
            jax.ShapeDtypeStruct((NP, 4), f32),
            jax.ShapeDtypeStruct((NP, 4), f32),
        ],
    )(h1A, h1B, W2, A2a, A2b)


_MESH = dict(core_axis_name="c", subcore_axis_name="s",
             num_cores=2, num_subcores=16)
_NO_LAYOUT = pltpu.CompilerParams(needs_layout_passes=False)
_DEN = ROWS_PER_TILE * 2


KW = 576
CH_W = EP // 16 // KW
NG_W = KW // 16


@functools.lru_cache(maxsize=None)
def _make_sc_w():
    f32 = jnp.float32
    i32 = jnp.int32
    mesh = plsc.VectorSubcoreMesh(**_MESH)

    def body(sidx_h, didx_h, adcA_h, adcB_h, w_hbm, den_hbm,
             sidx, didx, adc, wq2, didx2, denb, den_sp):
        c = lax.axis_index("c")
        s = lax.axis_index("s")
        lane = lax.iota(i32, 16)
        zero16 = jnp.zeros((16,), f32)

        @pl.when(c == 0)
        def _():
            pltpu.sync_copy(adcA_h, adc)

        @pl.when(c == 1)
        def _():
            pltpu.sync_copy(adcB_h, adc)

        def zb(i, _):
            denb[pl.ds(16 * i, 16)] = zero16
            return 0
        lax.fori_loop(0, _DEN // 16, zb, 0)
        pltpu.sync_copy(denb, den_sp.at[pl.ds(s * _DEN, _DEN)])
        plsc.subcore_barrier()

        ebase = s * (CH_W * KW)

        def chunk(i, _):
            base = ebase + i * KW
            pltpu.sync_copy(sidx_h.at[pl.ds(base, KW)], sidx)
            pltpu.sync_copy(didx_h.at[pl.ds(base, KW)], didx)

            def grp(g, _):
                off = 16 * g
                sv = sidx[pl.ds(off, 16)]
                dv = didx[pl.ds(off, 16)]
                sv4 = sv * 4
                dv4 = dv * 4
                asA = plsc.load_gather(adc, [sv4])
                asB = plsc.load_gather(adc, [sv4 + 1])
                adA = plsc.load_gather(adc, [dv4 + 2])
                adB = plsc.load_gather(adc, [dv4 + 3])
                aA = asA + adA
                aB = asB + adB
                wA = jnp.exp(jnp.maximum(aA, 0.2 * aA))
                wB = jnp.exp(jnp.maximum(aB, 0.2 * aB))
                pos = lane * 2 + 32 * g
                dv2 = dv * 2
                plsc.store_scatter(wq2, [pos], wA)
                plsc.store_scatter(wq2, [pos + 1], wB)
                plsc.store_scatter(didx2, [pos], dv2)
                plsc.store_scatter(didx2, [pos + 1], dv2 + 1)
                return 0
            lax.fori_loop(0, NG_W, grp, 0)

            pltpu.sync_copy(wq2, den_sp.at[didx2], add=True)
            pltpu.sync_copy(wq2, w_hbm.at[pl.ds(c * (EP * 2) + base * 2, KW * 2)])
            return 0
        lax.fori_loop(0, CH_W, chunk, 0)
        plsc.subcore_barrier()

        pltpu.sync_copy(den_sp.at[pl.ds(s * _DEN, _DEN)], denb)
        pltpu.sync_copy(denb, den_hbm.at[pl.ds(c * (NP * 2) + s * _DEN, _DEN)])

    scratch = [
        pltpu.VMEM((KW,), i32),
        pltpu.VMEM((KW,), i32),
        pltpu.VMEM((NP * 4,), f32),
        pltpu.VMEM((KW * 2,), f32),
        pltpu.VMEM((KW * 2,), i32),
        pltpu.VMEM((_DEN,), f32),
        pltpu.VMEM_SHARED((NP * 2,), f32),
    ]
    out_type = (jax.ShapeDtypeStruct((2 * EP * 2,), f32),
                jax.ShapeDtypeStruct((2 * NP * 2,), f32))
    return pl.kernel(body, out_type=out_type, mesh=mesh, scratch_types=scratch,
                     compiler_params=_NO_LAYOUT)


@functools.lru_cache(maxsize=None)
def _make_sc_m(C_OUT, NVA, do_relu):
    f32 = jnp.float32
    i32 = jnp.int32
    NVOUT = C_OUT // 16
    mesh = plsc.VectorSubcoreMesh(**_MESH)

    def body(sidx_h, didx_h, tabA, tabB, w_hbm, den_hbm, bias_h,
             outA, outB,
             sidx, didx, wq2, grow, fout, bvec, denb, acc_sp, sem):
        c = lax.axis_index("c")
        s = lax.axis_index("s")
        row0 = s * ROWS_PER_TILE
        zero16 = jnp.zeros((16,), f32)

        pltpu.sync_copy(bias_h.at[pl.ds(8 * c, 8)], bvec)

        def zrow(e, _):
            for v in range(8):
                grow[e, pl.ds(16 * v, 16)] = zero16
            return 0
        lax.fori_loop(0, K, zrow, 0)
        pltpu.sync_copy(grow, acc_sp.at[pl.ds(row0, K)])
        pltpu.sync_copy(grow, acc_sp.at[pl.ds(row0 + K, K)])
        pltpu.sync_copy(grow.at[pl.ds(0, ROWS_PER_TILE - 2 * K)],
                        acc_sp.at[pl.ds(row0 + 2 * K, ROWS_PER_TILE - 2 * K)])
        plsc.subcore_barrier()

        ebase = s * (CHUNKS_PER_TILE * K)

        def chunk(i, _):
            base = ebase + i * K
            pltpu.sync_copy(sidx_h.at[pl.ds(base, K)], sidx)
            pltpu.sync_copy(didx_h.at[pl.ds(base, K)], didx)
            pltpu.sync_copy(w_hbm.at[pl.ds(c * (EP * 2) + base * 2, K * 2)],
                            wq2)

            @pl.when(c == 0)
            def _():
                pltpu.async_copy(tabA.at[sidx], grow, sem).wait()

            @pl.when(c == 1)
            def _():
                pltpu.async_copy(tabB.at[sidx], grow, sem).wait()

            def srow(e, _):
                e2 = jnp.full((16,), 0, i32) + 2 * e
                wa = plsc.load_gather(wq2, [e2])
                wb = plsc.load_gather(wq2, [e2 + 1])
                for v in range(8):
                    w = wa if v < NVA else wb
                    grow[e, pl.ds(16 * v, 16)] = grow[e, pl.ds(16 * v, 16)] * w
                return 0
            lax.fori_loop(0, K, srow, 0)

            pltpu.sync_copy(grow, acc_sp.at[didx], add=True)
            return 0
        lax.fori_loop(0, CHUNKS_PER_TILE, chunk, 0)
        plsc.subcore_barrier()

        def fin(half, _):
            r0 = row0 + half * FSZ
            pltpu.sync_copy(acc_sp.at[pl.ds(r0, FSZ)], grow.at[pl.ds(0, FSZ)])
            pltpu.sync_copy(
                den_hbm.at[pl.ds(c * (NP * 2) + r0 * 2, FSZ * 2)], denb)

            def frow(n, _):
                n2 = jnp.full((16,), 0, i32) + 2 * n
                da = plsc.load_gather(denb, [n2]) + 1e-16
                db = plsc.load_gather(denb, [n2 + 1]) + 1e-16
                for v in range(NVOUT):
                    d = da if v < NVA else db
                    xv = grow[n, pl.ds(16 * v, 16)] / d + bvec[0, pl.ds(16 * v, 16)]
                    if do_relu:
                        xv = jnp.maximum(xv, 0.0)
                    if C_OUT == 128:
                        grow[n, pl.ds(16 * v, 16)] = xv
                    else:
                        fout[n, pl.ds(16 * v, 16)] = xv
                return 0
            lax.fori_loop(0, FSZ, frow, 0)

            src = grow if C_OUT == 128 else fout

            @pl.when(c == 0)
            def _():
                pltpu.sync_copy(src.at[pl.ds(0, FSZ)], outA.at[pl.ds(r0, FSZ)])

            @pl.when(c == 1)
            def _():
                pltpu.sync_copy(src.at[pl.ds(0, FSZ)], outB.at[pl.ds(r0, FSZ)])
            return 0
        lax.fori_loop(0, ROWS_PER_TILE // FSZ, fin, 0)

    scratch = [
        pltpu.VMEM((K,), i32),
        pltpu.VMEM((K,), i32),
        pltpu.VMEM((K * 2,), f32),
        pltpu.VMEM((K, 128), f32),
        pltpu.VMEM((FSZ, C_OUT), f32) if C_OUT != 128
        else pltpu.VMEM((8, 128), f32),
        pltpu.VMEM((8, 128), f32),
        pltpu.VMEM((FSZ * 2,), f32),
        pltpu.VMEM_SHARED((NP, 128), f32),
        pltpu.SemaphoreType.DMA,
    ]
    out_type = (jax.ShapeDtypeStruct((NP, C_OUT), f32),
                jax.ShapeDtypeStruct((NP, C_OUT), f32))
    return pl.kernel(body, out_type=out_type, mesh=mesh, scratch_types=scratch,
                     compiler_params=_NO_LAYOUT)


CH_M2 = CHUNKS_PER_TILE // 2


@functools.lru_cache(maxsize=None)
def _make_sc_m2():
    f32 = jnp.float32
    i32 = jnp.int32
    mesh = plsc.VectorSubcoreMesh(**_MESH)

    def body(sidx_h, didx_h, tab, w_hbm, outX,
             sidx, didx, wqA, wqB, grow, acc_sp, sem):
        c = lax.axis_index("c")
        s = lax.axis_index("s")
        row0 = s * ROWS_PER_TILE
        zero16 = jnp.zeros((16,), f32)

        def zrow(e, _):
            for v in range(8):
                grow[e, pl.ds(16 * v, 16)] = zero16
            return 0
        lax.fori_loop(0, K, zrow, 0)
        pltpu.sync_copy(grow, acc_sp.at[pl.ds(row0, K)])
        pltpu.sync_copy(grow, acc_sp.at[pl.ds(row0 + K, K)])
        pltpu.sync_copy(grow.at[pl.ds(0, ROWS_PER_TILE - 2 * K)],
                        acc_sp.at[pl.ds(row0 + 2 * K, ROWS_PER_TILE - 2 * K)])
        plsc.subcore_barrier()

        ebase = (c * 16 + s) * (CH_M2 * K)

        def chunk(i, _):
            base = ebase + i * K
            pltpu.sync_copy(sidx_h.at[pl.ds(base, K)], sidx)
            pltpu.sync_copy(didx_h.at[pl.ds(base, K)], didx)
            pltpu.sync_copy(w_hbm.at[pl.ds(base * 2, K * 2)], wqA)
            pltpu.sync_copy(w_hbm.at[pl.ds(EP * 2 + base * 2, K * 2)], wqB)
            pltpu.async_copy(tab.at[sidx], grow, sem).wait()

            def srow(e, _):
                e2 = jnp.full((16,), 0, i32) + 2 * e
                w0 = plsc.load_gather(wqA, [e2])
                w1 = plsc.load_gather(wqA, [e2 + 1])
                w2 = plsc.load_gather(wqB, [e2])
                w3 = plsc.load_gather(wqB, [e2 + 1])
                ws = (w0, w0, w1, w1, w2, w2, w3, w3)
                for v in range(8):
                    grow[e, pl.ds(16 * v, 16)] = (
                        grow[e, pl.ds(16 * v, 16)] * ws[v])
                return 0
            lax.fori_loop(0, K, srow, 0)

            pltpu.sync_copy(grow, acc_sp.at[didx], add=True)
            return 0
        lax.fori_loop(0, CH_M2, chunk, 0)
        plsc.subcore_barrier()

        def fin(half, _):
            r0 = row0 + half * FSZ
            pltpu.sync_copy(acc_sp.at[pl.ds(r0, FSZ)], grow.at[pl.ds(0, FSZ)])
            pltpu.sync_copy(grow.at[pl.ds(0, FSZ)],
                            outX.at[pl.ds(c * NP + r0, FSZ)])
            return 0
        lax.fori_loop(0, ROWS_PER_TILE // FSZ, fin, 0)

    scratch = [
        pltpu.VMEM((K,), i32),
        pltpu.VMEM((K,), i32),
        pltpu.VMEM((K * 2,), f32),
        pltpu.VMEM((K * 2,), f32),
        pltpu.VMEM((K, 128), f32),
        pltpu.VMEM_SHARED((NP, 128), f32),
        pltpu.SemaphoreType.DMA,
    ]
    out_type = jax.ShapeDtypeStruct((2 * NP, 128), f32)
    return pl.kernel(body, out_type=out_type, mesh=mesh, scratch_types=scratch,
                     compiler_params=_NO_LAYOUT)


def _att_mat(att_s, att_d, width, g0):
    cols = []
    for att, g in ((att_s, g0), (att_s, g0 + 1), (att_d, g0), (att_d, g0 + 1)):
        sel = jnp.zeros((HEADS, 1), att.dtype).at[g, 0].set(1.0)
        cols.append((att[:, :, None] * sel[:, None, :]).reshape(width, 1))
    return jnp.concatenate(cols, axis=1)


def kernel(x, edge_index, W1, att_src1, att_dst1, b1, W2, att_src2, att_dst2,
           b2, Wres, bres):
    i32 = jnp.int32
    loop = jnp.arange(N_NODES, dtype=i32)
    pad = jnp.full((EP - E_REAL,), N_NODES, dtype=i32)
    src = jnp.concatenate([edge_index[0].astype(i32), loop, pad])
    dst = jnp.concatenate([edge_index[1].astype(i32), loop, pad])

    x_p = jnp.pad(x, ((0, NP - N_NODES), (0, 0)))
    A1a = _att_mat(att_src1, att_dst1, HID_DIM, 0)
    A1b = _att_mat(att_src1, att_dst1, HID_DIM, 2)
    A2a = _att_mat(att_src2, att_dst2, OUT_DIM, 0)
    A2b = _att_mat(att_src2, att_dst2, OUT_DIM, 2)

    b1p = jnp.zeros((16, 128), jnp.float32)
    b1p = b1p.at[0].set(b1[:128]).at[8].set(b1[128:])

    tabA, tabB, adc1A, adc1B, res = _tc1(
        x_p, W1, Wres, A1a, A1b, bres.reshape(1, OUT_DIM))
    w1, den1 = _make_sc_w()(src, dst, adc1A.reshape(-1), adc1B.reshape(-1))
    h1A, h1B = _make_sc_m(128, 4, True)(
        src, dst, tabA, tabB, w1, den1, b1p)
    h2, adc2A, adc2B = _tc2(h1A, h1B, W2, A2a, A2b)
    w2, den2 = _make_sc_w()(src, dst, adc2A.reshape(-1), adc2B.reshape(-1))
    acc2 = _make_sc_m2()(src, dst, h2, w2)
    acc = acc2[:N_NODES] + acc2[NP:NP + N_NODES]
    den = den2.reshape(2, NP, 2)
    den_nh = jnp.concatenate([den[0], den[1]], axis=1)[:N_NODES]
    dd = jnp.repeat(den_nh + 1e-16, OUT_DIM // 4, axis=1)
    return acc / dd + b2 + res[:N_NODES]
